# baseline (speedup 1.0000x reference)
import jax
import jax.numpy as jnp
from jax import lax
from jax.experimental import pallas as pl
from jax.experimental.pallas import tpu as pltpu
from jax.experimental.pallas import tpu_sc as plsc

N = 10000
D = 128
CD = 64
E = 320000
LANES = 128
CHUNK = 16
NSC = 2
NTILE = 16
NACC = 10240
RPT = NACC // NTILE
OPT = 624
TAIL = N - OPT * NTILE

_MESH = plsc.VectorSubcoreMesh(core_axis_name="c", subcore_axis_name="s",
                               num_cores=NSC, num_subcores=NTILE)


def _make_seg(nsteps):
    def body(tbl, gidx_h, sidx_h, out, gidx, sidx, rows, acc, gsem, ssem):
        c = lax.axis_index("c")
        s = lax.axis_index("s")
        wid = c * NTILE + s

        def zrow(i, _):
            r = i // (D // 16)
            k = (i % (D // 16)) * 16
            rows[0, r, pl.ds(k, 16)] = jnp.zeros((16,), jnp.float32)
            return 0
        lax.fori_loop(0, LANES * (D // 16), zrow, 0)
        base = s * RPT
        for j in range(RPT // LANES):
            pltpu.sync_copy(rows.at[0], acc.at[pl.ds(base + j * LANES, LANES)])
        plsc.subcore_barrier()

        pltpu.sync_copy(gidx_h.at[wid, pl.ds(0, CHUNK)], gidx.at[0])
        pltpu.sync_copy(sidx_h.at[wid, pl.ds(0, CHUNK)], sidx.at[0])
        pltpu.async_copy(tbl.at[gidx.at[0, 0]], rows.at[0], gsem)

        def step(j, _):
            pj = j % 2
            jn = j + 1
            bn = jn // CHUNK

            @pl.when((jn % CHUNK == 0) & (jn < nsteps))
            def _stage():
                bb = bn % 2
                pltpu.sync_copy(gidx_h.at[wid, pl.ds(bn * CHUNK, CHUNK)],
                                gidx.at[bb])
                pltpu.sync_copy(sidx_h.at[wid, pl.ds(bn * CHUNK, CHUNK)],
                                sidx.at[bb])

            bj = (j // CHUNK) % 2
            pltpu.make_async_copy(tbl.at[gidx.at[bj, j % CHUNK]],
                                  rows.at[pj], gsem).wait()
            pltpu.async_copy(rows.at[pj], acc.at[sidx.at[bj, j % CHUNK]],
                             ssem, add=True)

            @pl.when(jn < nsteps)
            def _next():
                @pl.when(j >= 1)
                def _ws():
                    pltpu.make_async_copy(rows.at[1 - pj],
                                          acc.at[sidx.at[0, 0]], ssem).wait()
                pltpu.async_copy(tbl.at[gidx.at[(jn // CHUNK) % 2,
                                                jn % CHUNK]],
                                 rows.at[1 - pj], gsem)
            return 0
        lax.fori_loop(0, nsteps, step, 0)
        pltpu.make_async_copy(rows.at[(nsteps - 1) % 2],
                              acc.at[sidx.at[0, 0]], ssem).wait()
        plsc.subcore_barrier()

        ob = s * OPT
        pltpu.sync_copy(acc.at[pl.ds(ob, OPT)], out.at[c, pl.ds(ob, OPT)])
        tb = OPT * NTILE

        @pl.when(s == 0)
        def _tail():
            pltpu.sync_copy(acc.at[pl.ds(tb, TAIL)], out.at[c, pl.ds(tb, TAIL)])

    return pl.kernel(
        body,
        out_type=jax.ShapeDtypeStruct((NSC, N, D), jnp.float32),
        mesh=_MESH,
        scratch_types=[
            pltpu.VMEM((2, CHUNK, LANES), jnp.int32),
            pltpu.VMEM((2, CHUNK, LANES), jnp.int32),
            pltpu.VMEM((2, LANES, D), jnp.float32),
            pltpu.VMEM_SHARED((NACC, D), jnp.float32),
            pltpu.SemaphoreType.DMA,
            pltpu.SemaphoreType.DMA,
        ])


def _make_cnt(nsteps):
    nblk = nsteps // CHUNK

    def body(sidx_h, out, sidx, ones_v, cacc):
        c = lax.axis_index("c")
        s = lax.axis_index("s")
        wid = c * NTILE + s

        def zrow(i, _):
            r = i // (D // 16)
            k = (i % (D // 16)) * 16
            ones_v[r, pl.ds(k, 16)] = jnp.zeros((16,), jnp.float32)
            return 0
        lax.fori_loop(0, LANES * (D // 16), zrow, 0)
        base = s * RPT
        for j in range(RPT // LANES):
            pltpu.sync_copy(ones_v, cacc.at[pl.ds(base + j * LANES, LANES)])

        def onerow(i, _):
            r = i // (D // 16)
            k = (i % (D // 16)) * 16
            ones_v[r, pl.ds(k, 16)] = jnp.ones((16,), jnp.float32)
            return 0
        lax.fori_loop(0, LANES * (D // 16), onerow, 0)
        plsc.subcore_barrier()

        def blk_body(b, _):
            pltpu.sync_copy(sidx_h.at[wid, pl.ds(b * CHUNK, CHUNK)], sidx)

            def step(j, _):
                pltpu.sync_copy(ones_v, cacc.at[sidx.at[j]], add=True)
                return 0
            lax.fori_loop(0, CHUNK, step, 0)
            return 0
        lax.fori_loop(0, nblk, blk_body, 0)
        plsc.subcore_barrier()

        ob = s * OPT
        pltpu.sync_copy(cacc.at[pl.ds(ob, OPT)], out.at[c, pl.ds(ob, OPT)])
        tb = OPT * NTILE

        @pl.when(s == 0)
        def _tail():
            pltpu.sync_copy(cacc.at[pl.ds(tb, TAIL)],
                            out.at[c, pl.ds(tb, TAIL)])

    return pl.kernel(
        body,
        out_type=jax.ShapeDtypeStruct((NSC, N, D), jnp.float32),
        mesh=_MESH,
        scratch_types=[
            pltpu.VMEM((CHUNK, LANES), jnp.int32),
            pltpu.VMEM((LANES, D), jnp.float32),
            pltpu.VMEM_SHARED((NACC, D), jnp.float32),
        ])


_seg_a = _make_seg(80)
_seg_c = _make_seg(160)
_cnt_k = _make_cnt(160)


BN = 2000


def _dotT(a, w):
    return lax.dot_general(a, w, (((1,), (1,)), ((), ())),
                           preferred_element_type=jnp.float32)


def _enc_body(x, w, b, o):
    v = jnp.maximum(_dotT(x[...], w[...]) + b[...], 0.0)
    o[0] = v
    o[1] = v


_enc = pl.pallas_call(
    _enc_body,
    grid=(N // BN,),
    in_specs=[pl.BlockSpec((BN, D), lambda i: (i, 0)),
              pl.BlockSpec((D, D), lambda i: (0, 0)),
              pl.BlockSpec((1, D), lambda i: (0, 0))],
    out_specs=pl.BlockSpec((2, BN, D), lambda i: (0, i, 0)),
    out_shape=jax.ShapeDtypeStruct((2, N, D), jnp.float32),
)


def _b_body(sums, cnt, x, wl, bl, wr, gwl, gbl, gwr, xo, gout):
    sm = sums[...]
    denom = jnp.maximum(cnt[0, :, 0:1], 1.0)
    agg = (sm[0] + sm[1]) / denom
    xx = x[0]
    xo[...] = jnp.maximum(_dotT(agg, wl[...]) + bl[...] + _dotT(xx, wr[...]),
                          0.0)
    g = jnp.maximum(_dotT(agg, gwl[...]) + gbl[...] + _dotT(xx, gwr[...]),
                    0.0)
    gout[0] = g
    gout[1] = g * g


_bk = pl.pallas_call(
    _b_body,
    grid=(N // BN,),
    in_specs=[pl.BlockSpec((2, BN, D), lambda i: (0, i, 0)),
              pl.BlockSpec((2, BN, D), lambda i: (0, i, 0)),
              pl.BlockSpec((1, BN, D), lambda i: (0, i, 0)),
              pl.BlockSpec((D, D), lambda i: (0, 0)),
              pl.BlockSpec((1, D), lambda i: (0, 0)),
              pl.BlockSpec((D, D), lambda i: (0, 0)),
              pl.BlockSpec((D, D), lambda i: (0, 0)),
              pl.BlockSpec((1, D), lambda i: (0, 0)),
              pl.BlockSpec((D, D), lambda i: (0, 0))],
    out_specs=[pl.BlockSpec((BN, D), lambda i: (i, 0)),
               pl.BlockSpec((2, BN, D), lambda i: (0, i, 0))],
    out_shape=[jax.ShapeDtypeStruct((N, D), jnp.float32),
               jax.ShapeDtypeStruct((2, N, D), jnp.float32)],
)


def _d_body(x, xn, xg, s12, cnt, o):
    c0 = cnt[1, :, 0:1]
    g = xg[0]
    sm = s12[...]
    dsum = c0 * g * g - 2.0 * g * sm[0] + sm[1]
    tau = jnp.tanh(dsum / jnp.maximum(c0, 1.0))
    xx = x[0]
    v = xx + tau * (xn[...] - xx)
    o[0] = v
    o[1] = v


_dk = pl.pallas_call(
    _d_body,
    grid=(N // BN,),
    in_specs=[pl.BlockSpec((1, BN, D), lambda i: (0, i, 0)),
              pl.BlockSpec((BN, D), lambda i: (i, 0)),
              pl.BlockSpec((1, BN, D), lambda i: (0, i, 0)),
              pl.BlockSpec((2, BN, D), lambda i: (0, i, 0)),
              pl.BlockSpec((2, BN, D), lambda i: (0, i, 0))],
    out_specs=pl.BlockSpec((2, BN, D), lambda i: (0, i, 0)),
    out_shape=jax.ShapeDtypeStruct((2, N, D), jnp.float32),
)


def _dec_body(x, w, b, o):
    o[...] = _dotT(x[0], w[...]) + b[...]


_dec = pl.pallas_call(
    _dec_body,
    grid=(N // BN,),
    in_specs=[pl.BlockSpec((1, BN, D), lambda i: (0, i, 0)),
              pl.BlockSpec((CD, D), lambda i: (0, 0)),
              pl.BlockSpec((1, CD), lambda i: (0, 0))],
    out_specs=pl.BlockSpec((BN, CD), lambda i: (i, 0)),
    out_shape=jax.ShapeDtypeStruct((N, CD), jnp.float32),
)


def kernel(x, edge_index, enc_W, enc_b, conv_Wl, conv_bl, conv_Wr,
           gg_Wl, gg_bl, gg_Wr, dec_W, dec_b):
    EPAD = 32 * 80 * LANES
    PAD = EPAD - E
    src = edge_index[0]
    dst = edge_index[1]
    pz = jnp.zeros((PAD,), jnp.int32)
    pt = N + (jnp.arange(PAD, dtype=jnp.int32) % (NACC - N))
    src_g = jnp.concatenate([src, pz])
    dst_t = jnp.concatenate([dst, pt])
    src_g32 = src_g.reshape(32, 80, LANES)
    gA = jnp.concatenate([src_g32[:16], src_g32[16:] + N], axis=0)
    sA = dst_t.reshape(32, 80, LANES)
    dst_g16 = jnp.concatenate([dst, pz]).reshape(16, 160, LANES)
    gC = jnp.concatenate([dst_g16, dst_g16 + N], axis=0)
    src_s16 = jnp.concatenate([src, pt]).reshape(16, 160, LANES)
    sC = jnp.concatenate([src_s16, src_s16], axis=0)
    sCnt = jnp.concatenate([dst_t.reshape(16, 160, LANES),
                            src_s16], axis=0)

    eb = enc_b.reshape(1, D)
    cbl = conv_bl.reshape(1, D)
    gbl = gg_bl.reshape(1, D)
    db = dec_b.reshape(1, CD)

    cnt2 = _cnt_k(sCnt)
    X = _enc(x, enc_W, eb)

    for _ in range(2):
        sumsA = _seg_a(X.reshape(2 * N, D), gA, sA)
        X_, G = _bk(sumsA, cnt2, X, conv_Wl, cbl, conv_Wr,
                    gg_Wl, gbl, gg_Wr)
        s12 = _seg_c(G.reshape(2 * N, D), gC, sC)
        X = _dk(X, X_, G, s12, cnt2)

    return _dec(X, dec_W, db)


# speedup vs baseline: 1.0481x; 1.0481x over previous
import jax
import jax.numpy as jnp
from jax import lax
from jax.experimental import pallas as pl
from jax.experimental.pallas import tpu as pltpu
from jax.experimental.pallas import tpu_sc as plsc

N = 10000
D = 128
CD = 64
E = 320000
LANES = 128
CHUNK = 16
NSC = 2
NTILE = 16
NACC = 10240
RPT = NACC // NTILE
OPT = 624
TAIL = N - OPT * NTILE

_MESH = plsc.VectorSubcoreMesh(core_axis_name="c", subcore_axis_name="s",
                               num_cores=NSC, num_subcores=NTILE)


def _make_seg(nsteps):
    def body(tbl, gidx_h, sidx_h, out, gidx, sidx, rows, acc, gsem, ssem):
        c = lax.axis_index("c")
        s = lax.axis_index("s")
        wid = c * NTILE + s

        def zrow(i, _):
            r = i // (D // 16)
            k = (i % (D // 16)) * 16
            rows[0, r, pl.ds(k, 16)] = jnp.zeros((16,), jnp.float32)
            return 0
        lax.fori_loop(0, LANES * (D // 16), zrow, 0)
        base = s * RPT
        for j in range(RPT // LANES):
            pltpu.sync_copy(rows.at[0], acc.at[pl.ds(base + j * LANES, LANES)])
        plsc.subcore_barrier()

        pltpu.sync_copy(gidx_h.at[wid, pl.ds(0, CHUNK)], gidx.at[0])
        pltpu.sync_copy(sidx_h.at[wid, pl.ds(0, CHUNK)], sidx.at[0])
        pltpu.async_copy(tbl.at[gidx.at[0, 0]], rows.at[0], gsem)

        def step(j, _):
            pj = j % 2
            jn = j + 1
            bn = jn // CHUNK

            @pl.when((jn % CHUNK == 0) & (jn < nsteps))
            def _stage():
                bb = bn % 2
                pltpu.sync_copy(gidx_h.at[wid, pl.ds(bn * CHUNK, CHUNK)],
                                gidx.at[bb])
                pltpu.sync_copy(sidx_h.at[wid, pl.ds(bn * CHUNK, CHUNK)],
                                sidx.at[bb])

            bj = (j // CHUNK) % 2

            @pl.when(jn < nsteps)
            def _next():
                @pl.when(j >= 1)
                def _ws():
                    pltpu.make_async_copy(rows.at[1 - pj],
                                          acc.at[sidx.at[0, 0]], ssem).wait()
                pltpu.async_copy(tbl.at[gidx.at[(jn // CHUNK) % 2,
                                                jn % CHUNK]],
                                 rows.at[1 - pj], gsem)

            pltpu.make_async_copy(tbl.at[gidx.at[bj, j % CHUNK]],
                                  rows.at[pj], gsem).wait()
            pltpu.async_copy(rows.at[pj], acc.at[sidx.at[bj, j % CHUNK]],
                             ssem, add=True)
            return 0
        lax.fori_loop(0, nsteps, step, 0)
        pltpu.make_async_copy(rows.at[(nsteps - 1) % 2],
                              acc.at[sidx.at[0, 0]], ssem).wait()
        plsc.subcore_barrier()

        ob = s * OPT
        pltpu.sync_copy(acc.at[pl.ds(ob, OPT)], out.at[c, pl.ds(ob, OPT)])
        tb = OPT * NTILE

        @pl.when(s == 0)
        def _tail():
            pltpu.sync_copy(acc.at[pl.ds(tb, TAIL)], out.at[c, pl.ds(tb, TAIL)])

    return pl.kernel(
        body,
        out_type=jax.ShapeDtypeStruct((NSC, N, D), jnp.float32),
        mesh=_MESH,
        scratch_types=[
            pltpu.VMEM((2, CHUNK, LANES), jnp.int32),
            pltpu.VMEM((2, CHUNK, LANES), jnp.int32),
            pltpu.VMEM((2, LANES, D), jnp.float32),
            pltpu.VMEM_SHARED((NACC, D), jnp.float32),
            pltpu.SemaphoreType.DMA,
            pltpu.SemaphoreType.DMA,
        ])


def _make_cnt(nsteps):
    nblk = nsteps // CHUNK

    def body(sidx_h, out, sidx, ones_v, cacc):
        c = lax.axis_index("c")
        s = lax.axis_index("s")
        wid = c * NTILE + s

        def zrow(i, _):
            r = i // (D // 16)
            k = (i % (D // 16)) * 16
            ones_v[r, pl.ds(k, 16)] = jnp.zeros((16,), jnp.float32)
            return 0
        lax.fori_loop(0, LANES * (D // 16), zrow, 0)
        base = s * RPT
        for j in range(RPT // LANES):
            pltpu.sync_copy(ones_v, cacc.at[pl.ds(base + j * LANES, LANES)])

        def onerow(i, _):
            r = i // (D // 16)
            k = (i % (D // 16)) * 16
            ones_v[r, pl.ds(k, 16)] = jnp.ones((16,), jnp.float32)
            return 0
        lax.fori_loop(0, LANES * (D // 16), onerow, 0)
        plsc.subcore_barrier()

        def blk_body(b, _):
            pltpu.sync_copy(sidx_h.at[wid, pl.ds(b * CHUNK, CHUNK)], sidx)

            def step(j, _):
                pltpu.sync_copy(ones_v, cacc.at[sidx.at[j]], add=True)
                return 0
            lax.fori_loop(0, CHUNK, step, 0)
            return 0
        lax.fori_loop(0, nblk, blk_body, 0)
        plsc.subcore_barrier()

        ob = s * OPT
        pltpu.sync_copy(cacc.at[pl.ds(ob, OPT)], out.at[c, pl.ds(ob, OPT)])
        tb = OPT * NTILE

        @pl.when(s == 0)
        def _tail():
            pltpu.sync_copy(cacc.at[pl.ds(tb, TAIL)],
                            out.at[c, pl.ds(tb, TAIL)])

    return pl.kernel(
        body,
        out_type=jax.ShapeDtypeStruct((NSC, N, D), jnp.float32),
        mesh=_MESH,
        scratch_types=[
            pltpu.VMEM((CHUNK, LANES), jnp.int32),
            pltpu.VMEM((LANES, D), jnp.float32),
            pltpu.VMEM_SHARED((NACC, D), jnp.float32),
        ])


_seg_a = _make_seg(80)
_seg_c = _make_seg(160)
_cnt_k = _make_cnt(160)


BN = 2000


def _dotT(a, w):
    return lax.dot_general(a, w, (((1,), (1,)), ((), ())),
                           preferred_element_type=jnp.float32)


def _enc_body(x, w, b, o):
    v = jnp.maximum(_dotT(x[...], w[...]) + b[...], 0.0)
    o[0] = v
    o[1] = v


_enc = pl.pallas_call(
    _enc_body,
    grid=(N // BN,),
    in_specs=[pl.BlockSpec((BN, D), lambda i: (i, 0)),
              pl.BlockSpec((D, D), lambda i: (0, 0)),
              pl.BlockSpec((1, D), lambda i: (0, 0))],
    out_specs=pl.BlockSpec((2, BN, D), lambda i: (0, i, 0)),
    out_shape=jax.ShapeDtypeStruct((2, N, D), jnp.float32),
)


def _b_body(sums, cnt, x, wl, bl, wr, gwl, gbl, gwr, xo, gout):
    sm = sums[...]
    denom = jnp.maximum(cnt[0, :, 0:1], 1.0)
    agg = (sm[0] + sm[1]) / denom
    xx = x[0]
    xo[...] = jnp.maximum(_dotT(agg, wl[...]) + bl[...] + _dotT(xx, wr[...]),
                          0.0)
    g = jnp.maximum(_dotT(agg, gwl[...]) + gbl[...] + _dotT(xx, gwr[...]),
                    0.0)
    gout[0] = g
    gout[1] = g * g


_bk = pl.pallas_call(
    _b_body,
    grid=(N // BN,),
    in_specs=[pl.BlockSpec((2, BN, D), lambda i: (0, i, 0)),
              pl.BlockSpec((2, BN, D), lambda i: (0, i, 0)),
              pl.BlockSpec((1, BN, D), lambda i: (0, i, 0)),
              pl.BlockSpec((D, D), lambda i: (0, 0)),
              pl.BlockSpec((1, D), lambda i: (0, 0)),
              pl.BlockSpec((D, D), lambda i: (0, 0)),
              pl.BlockSpec((D, D), lambda i: (0, 0)),
              pl.BlockSpec((1, D), lambda i: (0, 0)),
              pl.BlockSpec((D, D), lambda i: (0, 0))],
    out_specs=[pl.BlockSpec((BN, D), lambda i: (i, 0)),
               pl.BlockSpec((2, BN, D), lambda i: (0, i, 0))],
    out_shape=[jax.ShapeDtypeStruct((N, D), jnp.float32),
               jax.ShapeDtypeStruct((2, N, D), jnp.float32)],
)


def _d_body(x, xn, xg, s12, cnt, o):
    c0 = cnt[1, :, 0:1]
    g = xg[0]
    sm = s12[...]
    dsum = c0 * g * g - 2.0 * g * sm[0] + sm[1]
    tau = jnp.tanh(dsum / jnp.maximum(c0, 1.0))
    xx = x[0]
    v = xx + tau * (xn[...] - xx)
    o[0] = v
    o[1] = v


_dk = pl.pallas_call(
    _d_body,
    grid=(N // BN,),
    in_specs=[pl.BlockSpec((1, BN, D), lambda i: (0, i, 0)),
              pl.BlockSpec((BN, D), lambda i: (i, 0)),
              pl.BlockSpec((1, BN, D), lambda i: (0, i, 0)),
              pl.BlockSpec((2, BN, D), lambda i: (0, i, 0)),
              pl.BlockSpec((2, BN, D), lambda i: (0, i, 0))],
    out_specs=pl.BlockSpec((2, BN, D), lambda i: (0, i, 0)),
    out_shape=jax.ShapeDtypeStruct((2, N, D), jnp.float32),
)


def _dec_body(x, w, b, o):
    o[...] = _dotT(x[0], w[...]) + b[...]


_dec = pl.pallas_call(
    _dec_body,
    grid=(N // BN,),
    in_specs=[pl.BlockSpec((1, BN, D), lambda i: (0, i, 0)),
              pl.BlockSpec((CD, D), lambda i: (0, 0)),
              pl.BlockSpec((1, CD), lambda i: (0, 0))],
    out_specs=pl.BlockSpec((BN, CD), lambda i: (i, 0)),
    out_shape=jax.ShapeDtypeStruct((N, CD), jnp.float32),
)


def kernel(x, edge_index, enc_W, enc_b, conv_Wl, conv_bl, conv_Wr,
           gg_Wl, gg_bl, gg_Wr, dec_W, dec_b):
    EPAD = 32 * 80 * LANES
    PAD = EPAD - E
    src = edge_index[0]
    dst = edge_index[1]
    pz = jnp.zeros((PAD,), jnp.int32)
    pt = N + (jnp.arange(PAD, dtype=jnp.int32) % (NACC - N))
    src_g = jnp.concatenate([src, pz])
    dst_t = jnp.concatenate([dst, pt])
    src_g32 = src_g.reshape(32, 80, LANES)
    gA = jnp.concatenate([src_g32[:16], src_g32[16:] + N], axis=0)
    sA = dst_t.reshape(32, 80, LANES)
    dst_g16 = jnp.concatenate([dst, pz]).reshape(16, 160, LANES)
    gC = jnp.concatenate([dst_g16, dst_g16 + N], axis=0)
    src_s16 = jnp.concatenate([src, pt]).reshape(16, 160, LANES)
    sC = jnp.concatenate([src_s16, src_s16], axis=0)
    sCnt = jnp.concatenate([dst_t.reshape(16, 160, LANES),
                            src_s16], axis=0)

    eb = enc_b.reshape(1, D)
    cbl = conv_bl.reshape(1, D)
    gbl = gg_bl.reshape(1, D)
    db = dec_b.reshape(1, CD)

    cnt2 = _cnt_k(sCnt)
    X = _enc(x, enc_W, eb)

    for _ in range(2):
        sumsA = _seg_a(X.reshape(2 * N, D), gA, sA)
        X_, G = _bk(sumsA, cnt2, X, conv_Wl, cbl, conv_Wr,
                    gg_Wl, gbl, gg_Wr)
        s12 = _seg_c(G.reshape(2 * N, D), gC, sC)
        X = _dk(X, X_, G, s12, cnt2)

    return _dec(X, dec_W, db)
